# baseline (speedup 1.0000x reference)
import functools

import jax
import jax.numpy as jnp
from jax import lax
from jax.experimental import pallas as pl
from jax.experimental.pallas import tpu as pltpu
from jax.experimental.pallas import tpu_sc as plsc

NC = 2
NS = 16
NW = NC * NS
L = 16


def _newton_rsqrt(ns):
    i = plsc.bitcast(ns, jnp.int32)
    i = jnp.int32(0x5F3759DF) - (i >> 1)
    y = plsc.bitcast(i, jnp.float32)
    for _ in range(3):
        y = y * (1.5 - 0.5 * ns * y * y)
    return y


@functools.cache
def _build(n_i, n_j):
    it = n_i // 128
    NSPLIT = 8
    NB = 5
    n_units = n_j * NSPLIT
    per_w = n_units // NW
    icq = it // NSPLIT
    assert per_w * NW == n_units and icq * NSPLIT == it and per_w % NB == 0

    mesh = plsc.VectorSubcoreMesh(
        core_axis_name="c", subcore_axis_name="s", num_cores=NC,
        num_subcores=NS)

    @functools.partial(
        pl.kernel,
        out_type=jax.ShapeDtypeStruct((n_j * 4 * it, 128), jnp.float32),
        mesh=mesh,
        scratch_types=[
            pltpu.VMEM((48,), jnp.float32),
            pltpu.VMEM((48,), jnp.float32),
            pltpu.VMEM((48,), jnp.float32),
            pltpu.VMEM((64,), jnp.float32),
            pltpu.VMEM((NB, icq, 128), jnp.int32),
            pltpu.VMEM((NB, icq * 4, 128), jnp.float32),
        ] + [pltpu.SemaphoreType.DMA] * (2 * NB),
        compiler_params=pltpu.CompilerParams(needs_layout_passes=False),
    )
    def sc_kernel(w_hbm, x_hbm, out_hbm, wtab, sqb, tabs, trep, idx_v, out_v,
                  *sems):
        wid = lax.axis_index("s") * NC + lax.axis_index("c")
        iota = lax.iota(jnp.int32, L)
        isem = sems[:NB]
        osem = sems[NB:]

        pltpu.sync_copy(w_hbm, wtab)
        for q in range(3):
            v = wtab[pl.ds(q * L, L)]
            sqb[pl.ds(q * L, L)] = v * v
        for q in range(3):
            r4 = ((iota + q * L) >> 2) << 2
            ns = (plsc.load_gather(sqb, [r4])
                  + plsc.load_gather(sqb, [r4 + 1])
                  + plsc.load_gather(sqb, [r4 + 2])
                  + plsc.load_gather(sqb, [r4 + 3]))
            y = _newton_rsqrt(ns)
            scale = 1.0 / (ns * y + 1e-7)
            scale = jnp.where(ns > 1.0, scale, 1.0)
            tabs[pl.ds(q * L, L)] = wtab[pl.ds(q * L, L)] * scale

        for d in range(4):
            trep[pl.ds(d * L, L)] = plsc.load_gather(
                tabs, [jnp.minimum(iota * 4 + d, 47)])

        def idx_copy(t, b):
            u = wid * per_w + t
            j = u // NSPLIT
            e = u % NSPLIT
            return pltpu.make_async_copy(
                x_hbm.at[j >> 3, pl.ds(e * icq, icq), j & 7, :],
                idx_v.at[b], isem[b])

        def out_copy(t, b):
            u = wid * per_w + t
            j = u // NSPLIT
            e = u % NSPLIT
            return pltpu.make_async_copy(
                out_v.at[b],
                out_hbm.at[pl.ds((j * it + e * icq) * 4, icq * 4), :],
                osem[b])

        for b in range(NB - 1):
            idx_copy(b, b).start()

        def quad_body(i, carry):
            for b in range(NB):
                t = NB * i + b
                idx_copy(t, b).wait()
                if b == 0:
                    idx_copy(t + NB - 1, NB - 1).start()
                else:
                    @pl.when(i < per_w // NB - 1)
                    def _():
                        idx_copy(t + NB - 1, b - 1).start()

                @pl.when(i >= 1)
                def _():
                    out_copy(t - NB, b).wait()

                @plsc.parallel_loop(0, icq, unroll=4)
                def q_loop(q):
                    for g in range(8):
                        idx = idx_v[b, q, pl.ds(g * L, L)]
                        for d in range(4):
                            gv = plsc.load_gather(trep, [idx + d * L])
                            out_v[b, q * 4 + d, pl.ds(g * L, L)] = gv

                out_copy(t, b).start()
            return carry

        lax.fori_loop(0, per_w // NB, quad_body, 0)
        for b in range(NB):
            out_copy(per_w - NB + b, b).wait()

    return sc_kernel


@jax.jit
def _embed(x, weight):
    n_i, n_j = x.shape
    it = n_i // 128
    xv = (x.astype(jnp.int32).T
          .reshape(n_j // 8, 8, it, 128)
          .transpose(0, 2, 1, 3))
    wf = jnp.pad(weight.astype(jnp.float32).reshape(-1), (0, 8))
    out5 = _build(n_i, n_j)(wf, xv)
    out = (out5.reshape(n_j, it, 4, 128)
           .transpose(1, 3, 0, 2)
           .reshape(n_i, n_j, 4))
    return out


def kernel(x, weight):
    return _embed(x, weight)


# speedup vs baseline: 1.2976x; 1.2976x over previous
import functools

import jax
import jax.numpy as jnp
from jax import lax
from jax.experimental import pallas as pl
from jax.experimental.pallas import tpu as pltpu
from jax.experimental.pallas import tpu_sc as plsc

NC = 2
NS = 16
NW = NC * NS
L = 16


def _newton_rsqrt(ns):
    i = plsc.bitcast(ns, jnp.int32)
    i = jnp.int32(0x5F3759DF) - (i >> 1)
    y = plsc.bitcast(i, jnp.float32)
    for _ in range(3):
        y = y * (1.5 - 0.5 * ns * y * y)
    return y


@functools.cache
def _build(n_i, n_j):
    it = n_i // 128
    NSPLIT = 16
    NB = 5
    n_units = n_j * NSPLIT
    per_w = n_units // NW
    icq = it // NSPLIT
    assert per_w * NW == n_units and icq * NSPLIT == it and per_w % NB == 0

    mesh = plsc.VectorSubcoreMesh(
        core_axis_name="c", subcore_axis_name="s", num_cores=NC,
        num_subcores=NS)

    @functools.partial(
        pl.kernel,
        out_type=jax.ShapeDtypeStruct((n_j * 4 * it, 128), jnp.float32),
        mesh=mesh,
        scratch_types=[
            pltpu.VMEM((48,), jnp.float32),
            pltpu.VMEM((48,), jnp.float32),
            pltpu.VMEM((48,), jnp.float32),
            pltpu.VMEM((64,), jnp.float32),
            pltpu.VMEM((NB, icq, 128), jnp.int32),
            pltpu.VMEM((NB, icq * 4, 128), jnp.float32),
        ] + [pltpu.SemaphoreType.DMA] * (2 * NB),
        compiler_params=pltpu.CompilerParams(needs_layout_passes=False),
    )
    def sc_kernel(w_hbm, x_hbm, out_hbm, wtab, sqb, tabs, trep, idx_v, out_v,
                  *sems):
        wid = lax.axis_index("s") * NC + lax.axis_index("c")
        iota = lax.iota(jnp.int32, L)
        isem = sems[:NB]
        osem = sems[NB:]

        pltpu.sync_copy(w_hbm, wtab)
        for q in range(3):
            v = wtab[pl.ds(q * L, L)]
            sqb[pl.ds(q * L, L)] = v * v
        for q in range(3):
            r4 = ((iota + q * L) >> 2) << 2
            ns = (plsc.load_gather(sqb, [r4])
                  + plsc.load_gather(sqb, [r4 + 1])
                  + plsc.load_gather(sqb, [r4 + 2])
                  + plsc.load_gather(sqb, [r4 + 3]))
            y = _newton_rsqrt(ns)
            scale = 1.0 / (ns * y + 1e-7)
            scale = jnp.where(ns > 1.0, scale, 1.0)
            tabs[pl.ds(q * L, L)] = wtab[pl.ds(q * L, L)] * scale

        for d in range(4):
            trep[pl.ds(d * L, L)] = plsc.load_gather(
                tabs, [jnp.minimum(iota * 4 + d, 47)])

        def idx_copy(t, b):
            u = wid * per_w + t
            j = u // NSPLIT
            e = u % NSPLIT
            return pltpu.make_async_copy(
                x_hbm.at[j >> 3, pl.ds(e * icq, icq), j & 7, :],
                idx_v.at[b], isem[b])

        def out_copy(t, b):
            u = wid * per_w + t
            j = u // NSPLIT
            e = u % NSPLIT
            return pltpu.make_async_copy(
                out_v.at[b],
                out_hbm.at[pl.ds((j * it + e * icq) * 4, icq * 4), :],
                osem[b])

        for b in range(NB - 1):
            idx_copy(b, b).start()

        def quad_body(i, carry):
            for b in range(NB):
                t = NB * i + b
                idx_copy(t, b).wait()
                if b == 0:
                    idx_copy(t + NB - 1, NB - 1).start()
                else:
                    @pl.when(i < per_w // NB - 1)
                    def _():
                        idx_copy(t + NB - 1, b - 1).start()

                @pl.when(i >= 1)
                def _():
                    out_copy(t - NB, b).wait()

                @plsc.parallel_loop(0, icq, unroll=4)
                def q_loop(q):
                    for g in range(8):
                        idx = idx_v[b, q, pl.ds(g * L, L)]
                        for d in range(4):
                            gv = plsc.load_gather(trep, [idx + d * L])
                            out_v[b, q * 4 + d, pl.ds(g * L, L)] = gv

                out_copy(t, b).start()
            return carry

        lax.fori_loop(0, per_w // NB, quad_body, 0)
        for b in range(NB):
            out_copy(per_w - NB + b, b).wait()

    return sc_kernel


@jax.jit
def _embed(x, weight):
    n_i, n_j = x.shape
    it = n_i // 128
    xv = (x.astype(jnp.int32).T
          .reshape(n_j // 8, 8, it, 128)
          .transpose(0, 2, 1, 3))
    wf = jnp.pad(weight.astype(jnp.float32).reshape(-1), (0, 8))
    out5 = _build(n_i, n_j)(wf, xv)
    out = (out5.reshape(n_j, it, 4, 128)
           .transpose(1, 3, 0, 2)
           .reshape(n_i, n_j, 4))
    return out


def kernel(x, weight):
    return _embed(x, weight)
